# baseline (speedup 1.0000x reference)
import functools

import jax
import jax.numpy as jnp
from jax.experimental import pallas as pl

_TOP_K = 32


def _body(xr_ref, r_ref, w1t_ref, b1_ref, w2t_ref, b2_ref, w3t_ref, b3_ref,
          w4t_ref, b4_ref, out_ref, mask_ref, idx_ref, *, n):
    R = xr_ref.shape[0]
    blk = xr_ref[...].reshape(R, n, 4)
    x0 = blk[:, :, 0]
    x1 = blk[:, :, 1]
    x2 = blk[:, :, 2]
    x3 = blk[:, :, 3]
    d2 = jnp.sqrt((x0 * x0 + 1e-6) + (x1 * x1 + 1e-6))
    iota = jax.lax.broadcasted_iota(jnp.int32, (R, n), 1)
    row0 = pl.program_id(0) * R
    rowids = row0 + jax.lax.broadcasted_iota(jnp.int32, (R, 1), 0)

    work = d2
    idx_cols = []
    g0, g1, g2, g3 = [], [], [], []
    for _ in range(_TOP_K):
        m = jnp.min(work, axis=1, keepdims=True)
        idx_k = jnp.min(jnp.where(work == m, iota, n), axis=1, keepdims=True)
        oh = iota == idx_k
        ohf = oh.astype(jnp.float32)
        g0.append(jnp.sum(x0 * ohf, axis=1, keepdims=True))
        g1.append(jnp.sum(x1 * ohf, axis=1, keepdims=True))
        g2.append(jnp.sum(x2 * ohf, axis=1, keepdims=True))
        g3.append(jnp.sum(x3 * ohf, axis=1, keepdims=True))
        idx_cols.append(idx_k)
        work = jnp.where(oh, jnp.inf, work)

    idx = jnp.concatenate(idx_cols, axis=1)
    xg0 = jnp.concatenate(g0, axis=1)
    xg1 = jnp.concatenate(g1, axis=1)
    xg2 = jnp.concatenate(g2, axis=1)
    xg3 = jnp.concatenate(g3, axis=1)
    eye = (idx == rowids).astype(jnp.float32)
    dn = jnp.sqrt((xg0 * xg0 + 1e-4) + (xg1 * xg1 + 1e-4))
    rv = r_ref[0, 0]

    feats = jnp.stack([xg0, xg1, xg2, xg3, eye, dn - rv], axis=-1)
    feats = feats.reshape(R * _TOP_K, 6)
    h = jax.nn.relu(jnp.dot(feats, w1t_ref[...],
                            preferred_element_type=jnp.float32) + b1_ref[...])
    h = jax.nn.relu(jnp.dot(h, w2t_ref[...],
                            preferred_element_type=jnp.float32) + b2_ref[...])
    h = jax.nn.relu(jnp.dot(h, w3t_ref[...],
                            preferred_element_type=jnp.float32) + b3_ref[...])
    h = jnp.dot(h, w4t_ref[...], preferred_element_type=jnp.float32) + b4_ref[...]

    mask = (dn <= 1.0).astype(jnp.float32)
    out_ref[...] = h.reshape(R, _TOP_K) * mask
    mask_ref[...] = mask
    idx_ref[...] = idx


def kernel(x, r, W1, b1, W2, b2, W3, b3, W4, b4):
    n = x.shape[0]
    R = 8
    xr = x.reshape(n, n * 4)
    full = lambda shape: pl.BlockSpec(shape, lambda i: (0,) * len(shape))
    out, mask, idx = pl.pallas_call(
        functools.partial(_body, n=n),
        grid=(n // R,),
        in_specs=[
            pl.BlockSpec((R, n * 4), lambda i: (i, 0)),
            full((1, 1)),
            full(W1.T.shape), full((1, b1.shape[0])),
            full(W2.T.shape), full((1, b2.shape[0])),
            full(W3.T.shape), full((1, b3.shape[0])),
            full(W4.T.shape), full((1, b4.shape[0])),
        ],
        out_specs=[
            pl.BlockSpec((R, _TOP_K), lambda i: (i, 0)),
            pl.BlockSpec((R, _TOP_K), lambda i: (i, 0)),
            pl.BlockSpec((R, _TOP_K), lambda i: (i, 0)),
        ],
        out_shape=[
            jax.ShapeDtypeStruct((n, _TOP_K), jnp.float32),
            jax.ShapeDtypeStruct((n, _TOP_K), jnp.float32),
            jax.ShapeDtypeStruct((n, _TOP_K), jnp.int32),
        ],
    )(xr, r.reshape(1, 1), W1.T, b1.reshape(1, -1), W2.T, b2.reshape(1, -1),
      W3.T, b3.reshape(1, -1), W4.T, b4.reshape(1, -1))
    return (out[:, :, None], mask[:, :, None], idx)
